# Initial kernel scaffold; baseline (speedup 1.0000x reference)
import jax
import jax.numpy as jnp
from jax.experimental import pallas as pl


def kernel(x, positions, pos_table, W, b):
    raise NotImplementedError("write your pallas kernel here")


# speedup vs baseline: 3.0345x; 3.0345x over previous
import functools

import jax
import jax.numpy as jnp
from jax import lax
from jax.experimental import pallas as pl
from jax.experimental.pallas import tpu as pltpu
from jax.experimental.pallas import tpu_sc as plsc

_LANES = 128


def _sc_gather(pos_flat, table_flat, n, v, d):
    nw = 32
    per_w = n // nw
    ch = 1024
    n_chunks = per_w // ch
    mesh = plsc.VectorSubcoreMesh(core_axis_name="c", subcore_axis_name="s")

    @functools.partial(
        pl.kernel,
        mesh=mesh,
        out_type=jax.ShapeDtypeStruct((n * d,), jnp.float32),
        scratch_types=[
            pltpu.VMEM((v * d,), jnp.float32),
            pltpu.VMEM((ch,), jnp.int32),
            pltpu.VMEM((ch * d,), jnp.float32),
        ],
    )
    def gather_kernel(pos_hbm, table_hbm, out_hbm, table_v, idx_v, rows_v):
        wid = lax.axis_index("s") * 2 + lax.axis_index("c")
        base = wid * per_w
        pltpu.sync_copy(table_hbm, table_v)

        def chunk_body(s, carry):
            tok0 = base + s * ch

            pltpu.sync_copy(pos_hbm.at[pl.ds(tok0, ch)], idx_v)

            def grp_body(g, c2):
                vec = idx_v[pl.ds(g * 16, 16)] * d
                t0 = g * 16
                for j in range(16):
                    p = vec[j]
                    rows_v[pl.ds((t0 + j) * d, 16)] = table_v[pl.ds(p, 16)]
                    rows_v[pl.ds((t0 + j) * d + 16, 16)] = table_v[pl.ds(p + 16, 16)]
                return c2

            lax.fori_loop(0, ch // 16, grp_body, 0)
            pltpu.sync_copy(rows_v, out_hbm.at[pl.ds(tok0 * d, ch * d)])
            return carry

        lax.fori_loop(0, n_chunks, chunk_body, 0)

    return gather_kernel(pos_flat, table_flat)


def _tc_combine(x_flat, posemb_flat, wt, b2d):
    n, di = x_flat.shape
    de = wt.shape[1]
    tb = 4096

    def body(x_ref, pe_ref, wt_ref, b_ref, o_ref):
        o_ref[...] = (
            jnp.dot(x_ref[...], wt_ref[...], preferred_element_type=jnp.float32)
            + b_ref[...]
            + pe_ref[...]
        )

    return pl.pallas_call(
        body,
        grid=(n // tb,),
        in_specs=[
            pl.BlockSpec((tb, di), lambda i: (i, 0)),
            pl.BlockSpec((tb, de), lambda i: (i, 0)),
            pl.BlockSpec((di, de), lambda i: (0, 0)),
            pl.BlockSpec((1, de), lambda i: (0, 0)),
        ],
        out_specs=pl.BlockSpec((tb, de), lambda i: (i, 0)),
        out_shape=jax.ShapeDtypeStruct((n, de), jnp.float32),
    )(x_flat, posemb_flat, wt, b2d)


def kernel(x, positions, pos_table, W, b):
    bsz, seq, di = x.shape
    de = W.shape[0]
    n = bsz * seq
    x_flat = x.reshape(n, di)
    pos_flat = positions.reshape(n).astype(jnp.int32)
    posemb = _sc_gather(pos_flat, pos_table.reshape(-1), n, pos_table.shape[0], de)
    out = _tc_combine(x_flat, posemb.reshape(n, de), W.T, b.reshape(1, de))
    return out.reshape(bsz, seq, de)
